# baseline (speedup 1.0000x reference)
import functools

import jax
import jax.numpy as jnp
from jax import lax
from jax.experimental import pallas as pl
from jax.experimental.pallas import tpu as pltpu
from jax.experimental.pallas import tpu_sc as plsc

_N_USERS = 25000
_N_ITEMS = 25000
_N = _N_USERS + _N_ITEMS
_EMB = 32
_E = 1600000

_NC = 2
_NS = 16
_NW = _NC * _NS

_CH = 128
_ROWS_PER_TILE = 396
_E_PAD = _NW * _ROWS_PER_TILE * _CH
_SUP = 12
_NSUP = _ROWS_PER_TILE // _SUP
_NTRI = _SUP // 3
_ZROWS = 80
_NZBLK = _N // _ZROWS
_ZBLK_PER_TILE = 40

_mesh = plsc.VectorSubcoreMesh(core_axis_name="c", subcore_axis_name="s")


@functools.partial(
    pl.kernel,
    out_type=[
        jax.ShapeDtypeStruct((_N, _EMB), jnp.float32),
        jax.ShapeDtypeStruct((_N, _EMB), jnp.float32),
    ],
    mesh=_mesh,
    compiler_params=pltpu.CompilerParams(use_tc_tiling_on_sc=False, needs_layout_passes=False),
    scratch_types=[
        pltpu.VMEM_SHARED((_N, _EMB), jnp.float32),
        pltpu.VMEM((_SUP, _CH), jnp.int32),
        pltpu.VMEM((_SUP, _CH), jnp.int32),
        pltpu.VMEM((_SUP, _CH), jnp.float32),
        pltpu.VMEM((_SUP, _CH), jnp.int32),
        pltpu.VMEM((_SUP, _CH), jnp.int32),
        pltpu.VMEM((_SUP, _CH), jnp.float32),
        pltpu.VMEM((_ZROWS, _EMB), jnp.float32),
        pltpu.VMEM((_CH, _EMB), jnp.bfloat16),
        pltpu.VMEM((_CH, _EMB), jnp.bfloat16),
        pltpu.VMEM((_CH, _EMB), jnp.bfloat16),
        pltpu.VMEM((_CH, _EMB), jnp.float32),
        pltpu.VMEM((_CH, _EMB), jnp.float32),
        pltpu.VMEM((_CH, _EMB), jnp.float32),
        pltpu.SemaphoreType.DMA,
        pltpu.SemaphoreType.DMA,
        pltpu.SemaphoreType.DMA,
        pltpu.SemaphoreType.DMA,
        pltpu.SemaphoreType.DMA,
        pltpu.SemaphoreType.DMA,
        pltpu.SemaphoreType.DMA,
        pltpu.SemaphoreType.DMA,
    ],
)
def _layer(src_hbm, dst_hbm, w_hbm, table_hbm, out0, out1,
           acc, src_a, dst_a, w_a, src_b, dst_b, w_b, zbuf,
           g0, g1, g2, s0, s1, s2, sg0, sg1, sg2, ss0, ss1, ss2, sm, smb):
    cid = lax.axis_index("c")
    sid = lax.axis_index("s")

    zero = jnp.zeros((16,), jnp.float32)

    def _zrow(r, carry):
        zbuf[r, pl.ds(0, 16)] = zero
        zbuf[r, pl.ds(16, 16)] = zero
        return carry

    lax.fori_loop(0, _ZROWS, _zrow, 0)
    for k in range(_ZBLK_PER_TILE):
        b = sid + _NS * k

        @pl.when(b < _NZBLK)
        def _():
            pltpu.async_copy(zbuf, acc.at[pl.ds(b * _ZROWS, _ZROWS)], sm)
    for k in range(_ZBLK_PER_TILE):
        b = sid + _NS * k

        @pl.when(b < _NZBLK)
        def _():
            pltpu.make_async_copy(zbuf, acc.at[pl.ds(b * _ZROWS, _ZROWS)],
                                  sm).wait()
    plsc.subcore_barrier()

    wid = sid * _NC + cid
    row0 = wid * _ROWS_PER_TILE

    def _meta_fire(sup, src_m, dst_m, w_m, sem):
        srow = row0 + sup * _SUP
        pltpu.async_copy(src_hbm.at[pl.ds(srow, _SUP)], src_m, sem)
        pltpu.async_copy(dst_hbm.at[pl.ds(srow, _SUP)], dst_m, sem)
        pltpu.async_copy(w_hbm.at[pl.ds(srow, _SUP)], w_m, sem)

    def _meta_wait(sup, src_m, dst_m, w_m, sem):
        srow = row0 + sup * _SUP
        pltpu.make_async_copy(src_hbm.at[pl.ds(srow, _SUP)], src_m, sem).wait()
        pltpu.make_async_copy(dst_hbm.at[pl.ds(srow, _SUP)], dst_m, sem).wait()
        pltpu.make_async_copy(w_hbm.at[pl.ds(srow, _SUP)], w_m, sem).wait()

    def _process(src_m, dst_m, w_m):
        def _scale(j, gbuf, sbuf):
            def _blk(b, c2):
                wv = w_m[j, pl.ds(b * 16, 16)]
                r0 = b * 16
                for e in range(16):
                    w = wv[e]
                    r = r0 + e
                    lo, hi = plsc.unpack(gbuf[r, pl.ds(0, _EMB)],
                                         format=plsc.PackFormat.INTERLEAVED)
                    sbuf[r, pl.ds(0, 16)] = lo * w
                    sbuf[r, pl.ds(16, 16)] = hi * w
                return c2

            lax.fori_loop(0, _CH // 16, _blk, 0)

        def _g_start(j, gbuf, sem):
            pltpu.async_copy(table_hbm.at[src_m.at[j]], gbuf, sem)

        def _g_wait(j, gbuf, sem):
            pltpu.make_async_copy(table_hbm.at[src_m.at[j]], gbuf, sem).wait()

        def _s_start(j, sbuf, sem):
            pltpu.async_copy(sbuf, acc.at[dst_m.at[j]], sem, add=True)

        def _s_wait(j, sbuf, sem):
            pltpu.make_async_copy(sbuf, acc.at[dst_m.at[j]], sem).wait()

        gbufs = ((g0, sg0), (g1, sg1), (g2, sg2))
        sbufs = ((s0, ss0), (s1, ss1), (s2, ss2))
        for u in range(3):
            _g_start(u, *gbufs[u])
        for u in range(3):
            gb, gs = gbufs[u]
            sb, ssem = sbufs[u]
            _g_wait(u, gb, gs)
            _scale(u, gb, sb)
            _g_start(u + 3, gb, gs)
            _s_start(u, sb, ssem)

        def _tri(t, c2):
            for u in range(3):
                j = 3 * t + u
                gb, gs = gbufs[u]
                sb, ssem = sbufs[u]
                _g_wait(j, gb, gs)
                _s_wait(j - 3, sb, ssem)
                _scale(j, gb, sb)
                _g_start(j + 3, gb, gs)
                _s_start(j, sb, ssem)
            return c2

        lax.fori_loop(1, _NTRI - 1, _tri, 0)

        jl = _SUP - 3
        for u in range(3):
            gb, gs = gbufs[u]
            sb, ssem = sbufs[u]
            _g_wait(jl + u, gb, gs)
            _s_wait(jl + u - 3, sb, ssem)
            _scale(jl + u, gb, sb)
            _s_start(jl + u, sb, ssem)
        for u in range(3):
            sb, ssem = sbufs[u]
            _s_wait(jl + u, sb, ssem)

    _meta_fire(0, src_a, dst_a, w_a, sm)

    def _pair_body(i, carry):
        sup_a = 2 * i
        _meta_wait(sup_a, src_a, dst_a, w_a, sm)

        @pl.when(sup_a + 1 < _NSUP)
        def _():
            _meta_fire(sup_a + 1, src_b, dst_b, w_b, smb)

        _process(src_a, dst_a, w_a)

        @pl.when(sup_a + 1 < _NSUP)
        def _():
            _meta_wait(sup_a + 1, src_b, dst_b, w_b, smb)

            @pl.when(sup_a + 2 < _NSUP)
            def _():
                _meta_fire(sup_a + 2, src_a, dst_a, w_a, sm)

            _process(src_b, dst_b, w_b)

        return carry

    lax.fori_loop(0, (_NSUP + 1) // 2, _pair_body, 0)

    plsc.subcore_barrier()

    nflush = _N // _NS

    @pl.when(cid == 0)
    def _():
        pltpu.sync_copy(acc.at[pl.ds(sid * nflush, nflush)],
                        out0.at[pl.ds(sid * nflush, nflush)])

    @pl.when(cid == 1)
    def _():
        pltpu.sync_copy(acc.at[pl.ds(sid * nflush, nflush)],
                        out1.at[pl.ds(sid * nflush, nflush)])


_BLK = 400
_NBLK = _N // _BLK
_BLK_PER_W = 4


@functools.partial(
    pl.kernel,
    out_type=[
        jax.ShapeDtypeStruct((_N, _EMB), jnp.float32),
        jax.ShapeDtypeStruct((_N, _EMB), jnp.bfloat16),
    ],
    mesh=_mesh,
    compiler_params=pltpu.CompilerParams(use_tc_tiling_on_sc=False, needs_layout_passes=False),
    scratch_types=[
        pltpu.VMEM((_BLK, _EMB), jnp.float32),
        pltpu.VMEM((_BLK, _EMB), jnp.float32),
        pltpu.VMEM((_BLK, _EMB), jnp.bfloat16),
        pltpu.SemaphoreType.DMA,
    ],
)
def _add2(a_hbm, b_hbm, out, out_bf, abuf, bbuf, pbuf, sem):
    cid = lax.axis_index("c")
    sid = lax.axis_index("s")
    wid = sid * _NC + cid

    def _accum(r, c2):
        lo = abuf[r, pl.ds(0, 16)] + bbuf[r, pl.ds(0, 16)]
        hi = abuf[r, pl.ds(16, 16)] + bbuf[r, pl.ds(16, 16)]
        abuf[r, pl.ds(0, 16)] = lo
        abuf[r, pl.ds(16, 16)] = hi
        pbuf[r, pl.ds(0, _EMB)] = plsc.pack(
            lo, hi, format=plsc.PackFormat.INTERLEAVED)
        return c2

    for k in range(_BLK_PER_W):
        b = wid + _NW * k

        @pl.when(b < _NBLK)
        def _():
            off = b * _BLK
            pltpu.async_copy(a_hbm.at[pl.ds(off, _BLK)], abuf, sem)
            pltpu.async_copy(b_hbm.at[pl.ds(off, _BLK)], bbuf, sem)
            pltpu.make_async_copy(a_hbm.at[pl.ds(off, _BLK)], abuf, sem).wait()
            pltpu.make_async_copy(b_hbm.at[pl.ds(off, _BLK)], bbuf, sem).wait()
            lax.fori_loop(0, _BLK, _accum, 0, unroll=4)
            pltpu.async_copy(abuf, out.at[pl.ds(off, _BLK)], sem)
            pltpu.async_copy(pbuf, out_bf.at[pl.ds(off, _BLK)], sem)
            pltpu.make_async_copy(abuf, out.at[pl.ds(off, _BLK)], sem).wait()
            pltpu.make_async_copy(pbuf, out_bf.at[pl.ds(off, _BLK)],
                                  sem).wait()


@functools.partial(
    pl.kernel,
    out_type=jax.ShapeDtypeStruct((_N, _EMB), jnp.float32),
    mesh=_mesh,
    compiler_params=pltpu.CompilerParams(use_tc_tiling_on_sc=False, needs_layout_passes=False),
    scratch_types=[
        pltpu.VMEM((_BLK, _EMB), jnp.float32),
        pltpu.VMEM((_BLK, _EMB), jnp.float32),
        pltpu.VMEM((_BLK, _EMB), jnp.float32),
        pltpu.VMEM((_BLK, _EMB), jnp.float32),
        pltpu.SemaphoreType.DMA,
    ],
)
def _add4_mean(a_hbm, b_hbm, c_hbm, d_hbm, out, abuf, bbuf, cbuf, dbuf, sem):
    cid = lax.axis_index("c")
    sid = lax.axis_index("s")
    wid = sid * _NC + cid
    third = jnp.float32(1.0 / 3.0)

    def _accum(r, c2):
        lo = ((abuf[r, pl.ds(0, 16)] + bbuf[r, pl.ds(0, 16)])
              + (cbuf[r, pl.ds(0, 16)] + dbuf[r, pl.ds(0, 16)])) * third
        hi = ((abuf[r, pl.ds(16, 16)] + bbuf[r, pl.ds(16, 16)])
              + (cbuf[r, pl.ds(16, 16)] + dbuf[r, pl.ds(16, 16)])) * third
        abuf[r, pl.ds(0, 16)] = lo
        abuf[r, pl.ds(16, 16)] = hi
        return c2

    for k in range(_BLK_PER_W):
        b = wid + _NW * k

        @pl.when(b < _NBLK)
        def _():
            off = b * _BLK
            pltpu.async_copy(a_hbm.at[pl.ds(off, _BLK)], abuf, sem)
            pltpu.async_copy(b_hbm.at[pl.ds(off, _BLK)], bbuf, sem)
            pltpu.async_copy(c_hbm.at[pl.ds(off, _BLK)], cbuf, sem)
            pltpu.async_copy(d_hbm.at[pl.ds(off, _BLK)], dbuf, sem)
            pltpu.make_async_copy(a_hbm.at[pl.ds(off, _BLK)], abuf, sem).wait()
            pltpu.make_async_copy(b_hbm.at[pl.ds(off, _BLK)], bbuf, sem).wait()
            pltpu.make_async_copy(c_hbm.at[pl.ds(off, _BLK)], cbuf, sem).wait()
            pltpu.make_async_copy(d_hbm.at[pl.ds(off, _BLK)], dbuf, sem).wait()
            lax.fori_loop(0, _BLK, _accum, 0, unroll=4)
            pltpu.sync_copy(abuf, out.at[pl.ds(off, _BLK)])


def _to_packed_bf16(x):
    lo = x[:, : _EMB // 2]
    hi = x[:, _EMB // 2:]
    return jnp.stack([lo, hi], axis=-1).reshape(_N, _EMB).astype(jnp.bfloat16)


def kernel(edge_index, edge_weight, user_emb_w, item_emb_w):
    all0 = jnp.concatenate([user_emb_w, item_emb_w], axis=0)
    pad = _E_PAD - _E
    src = jnp.concatenate([edge_index[0], jnp.zeros((pad,), jnp.int32)])
    dst = jnp.concatenate([edge_index[1], jnp.zeros((pad,), jnp.int32)])
    w = jnp.concatenate([edge_weight, jnp.zeros((pad,), jnp.float32)])
    src2d = src.reshape(_E_PAD // _CH, _CH)
    dst2d = dst.reshape(_E_PAD // _CH, _CH)
    w2d = w.reshape(_E_PAD // _CH, _CH)

    p0, p1 = _layer(src2d, dst2d, w2d, _to_packed_bf16(all0))
    emb1, emb1_bf = _add2(p0, p1)
    q0, q1 = _layer(src2d, dst2d, w2d, emb1_bf)
    final = _add4_mean(all0, emb1, q0, q1)
    return final[:_N_USERS], final[_N_USERS:]
